# baseline (speedup 1.0000x reference)
import functools

import jax
import jax.numpy as jnp
from jax import lax
from jax.experimental import pallas as pl
from jax.experimental.pallas import tpu as pltpu
from jax.experimental.pallas import tpu_sc as plsc

N = 10000
B = 32
IN_DIM = 2
HID = 32
OUT = 32
INPUT_SIZE = IN_DIM + HID
ISP = 36
WID_PAD = ISP * B
E = 160000
C = 9
F = WID_PAD // C
NTILES = 16
EB = 48
EPT = 10080
NB = EPT // EB
W = 42
NWIN = NB // W
NGRP = W // 3
E_PAD = EPT * NTILES
N_PAD = 10240
RPT = N_PAD // NTILES

_MESH = plsc.VectorSubcoreMesh(core_axis_name="c", subcore_axis_name="s")


def _scale(buf, evv, jl):
    def gi_body(gi, _):
        vv = evv[jl, pl.ds(16 * gi, 16)]
        for k in range(16):
            s = vv[k]
            e = 16 * gi + k
            for q in range(F // 16):
                buf[e, pl.ds(16 * q, 16)] = buf[e, pl.ds(16 * q, 16)] * s
        return 0
    lax.fori_loop(0, EB // 16, gi_body, 0)


def _make_spmm(shared_src: bool):
    @functools.partial(
        pl.kernel,
        mesh=_MESH,
        out_type=jax.ShapeDtypeStruct((2, N_PAD, WID_PAD), jnp.float32),
        scratch_types=[
            pltpu.VMEM((W, 2, EB), jnp.int32),
            pltpu.VMEM((W, EB), jnp.float32),
            pltpu.VMEM((EB, F), jnp.float32),
            pltpu.VMEM((EB, F), jnp.float32),
            pltpu.VMEM((EB, F), jnp.float32),
            pltpu.VMEM_SHARED((N_PAD, F), jnp.float32),
            pltpu.SemaphoreType.DMA,
            pltpu.SemaphoreType.DMA,
            pltpu.SemaphoreType.DMA,
            pltpu.SemaphoreType.DMA,
            pltpu.SemaphoreType.DMA,
            pltpu.SemaphoreType.DMA,
        ],
    )
    def spmm(src, eidx, evals, zeros, out,
             evw, evv, b0, b1, b2, acc, sg0, sg1, sg2, ss0, ss1, ss2):
        cid = lax.axis_index("c")
        sid = lax.axis_index("s")
        rowlo = sid * RPT
        bufs = (b0, b1, b2)
        sgs = (sg0, sg1, sg2)
        sss = (ss0, ss1, ss2)

        def chunk_body(c, _):
            coff = pl.multiple_of(c * F, F)
            if shared_src:
                src_c = src.at[:, pl.ds(coff, F)]
            else:
                src_c = src.at[cid, :, pl.ds(coff, F)]

            def start_gather(jl, slot):
                pltpu.async_copy(src_c.at[evw.at[jl, 1]], bufs[slot], sgs[slot])

            def wait_gather(jl, slot):
                pltpu.make_async_copy(
                    src_c.at[evw.at[jl, 1]], bufs[slot], sgs[slot]).wait()

            def start_scatter(jl, slot):
                pltpu.async_copy(bufs[slot], acc.at[evw.at[jl, 0]],
                                 sss[slot], add=True)

            def wait_scatter(jl, slot):
                pltpu.make_async_copy(
                    bufs[slot], acc.at[evw.at[jl, 0]], sss[slot]).wait()

            pltpu.sync_copy(zeros.at[pl.ds(rowlo, RPT)], acc.at[pl.ds(rowlo, RPT)])
            plsc.subcore_barrier()

            def win_body(w, _):
                pltpu.sync_copy(eidx.at[cid, sid, w], evw)
                pltpu.sync_copy(evals.at[cid, sid, w], evv)
                start_gather(0, 0)
                start_gather(1, 1)
                wait_gather(0, 0)
                _scale(b0, evv, 0)
                start_scatter(0, 0)
                start_gather(2, 2)
                wait_gather(1, 1)
                _scale(b1, evv, 1)
                start_scatter(1, 1)
                wait_scatter(0, 0)
                start_gather(3, 0)
                wait_gather(2, 2)
                _scale(b2, evv, 2)
                start_scatter(2, 2)

                def group_body(g, _):
                    for p in range(3):
                        jl = 3 * g + p
                        slot_next = (p + 1) % 3
                        wait_scatter(jl - 2, slot_next)
                        if p < 2:
                            start_gather(jl + 1, slot_next)
                        else:
                            @pl.when(g < NGRP - 1)
                            def _():
                                start_gather(jl + 1, slot_next)
                        wait_gather(jl, p)
                        _scale(bufs[p], evv, jl)
                        start_scatter(jl, p)
                    return 0

                lax.fori_loop(1, NGRP, group_body, 0)
                wait_scatter(W - 2, (W - 2) % 3)
                wait_scatter(W - 1, (W - 1) % 3)
                return 0

            lax.fori_loop(0, NWIN, win_body, 0)
            plsc.subcore_barrier()
            pltpu.sync_copy(acc.at[pl.ds(rowlo, RPT)],
                            out.at[cid, pl.ds(rowlo, RPT), pl.ds(coff, F)])
            plsc.subcore_barrier()
            return 0

        lax.fori_loop(0, C, chunk_body, 0)

    return spmm


_spmm_shared = _make_spmm(True)
_spmm_split = _make_spmm(False)

RB = 8000


def _mm_body(x0r, x1r, x2r, x3r, x4r, wr, br, outr):
    wb = wr[...].astype(jnp.bfloat16)
    acc = jnp.dot(x0r[...].astype(jnp.bfloat16), wb[0:ISP],
                  preferred_element_type=jnp.float32)
    acc += jnp.dot(x1r[...].astype(jnp.bfloat16), wb[ISP:2 * ISP],
                   preferred_element_type=jnp.float32)
    acc += jnp.dot(x2r[...].astype(jnp.bfloat16), wb[2 * ISP:3 * ISP],
                   preferred_element_type=jnp.float32)
    acc += jnp.dot(x3r[...].astype(jnp.bfloat16), wb[3 * ISP:4 * ISP],
                   preferred_element_type=jnp.float32)
    acc += jnp.dot(x4r[...].astype(jnp.bfloat16), wb[4 * ISP:5 * ISP],
                   preferred_element_type=jnp.float32)
    outr[...] = acc + br[...]


def _matmul(xs, wcat, bias2):
    m = N * B
    x_spec = pl.BlockSpec((RB, ISP), lambda i: (i, 0))
    return pl.pallas_call(
        _mm_body,
        grid=(m // RB,),
        in_specs=[x_spec] * 5 + [
            pl.BlockSpec((5 * ISP, OUT), lambda i: (0, 0)),
            pl.BlockSpec((1, OUT), lambda i: (0, 0)),
        ],
        out_specs=pl.BlockSpec((RB, OUT), lambda i: (i, 0)),
        out_shape=jax.ShapeDtypeStruct((m, OUT), jnp.float32),
    )(*xs, wcat, bias2)


def _pad_edges(idx, val):
    npad = E_PAD - E
    prow = N + (jnp.arange(npad, dtype=jnp.int32) % (N_PAD - N))
    pcol = jnp.arange(npad, dtype=jnp.int32) % N
    rows = jnp.concatenate([idx[0], prow])
    cols = jnp.concatenate([idx[1], pcol])
    vals = jnp.concatenate([val, jnp.zeros((npad,), jnp.float32)])
    return rows, cols, vals


def kernel(inputs, state, supp0_idx, supp0_val, supp1_idx, supp1_val,
           mu_weight, mu_biases, log_sigma_weight, log_sigma_biases,
           weight_noise, bias_noise):
    inp = inputs.reshape(B, N, IN_DIM)
    st = state.reshape(B, N, HID)
    cat = jnp.concatenate([inp, st], axis=2)
    catp = jnp.pad(cat, ((0, 0), (0, 0), (0, ISP - INPUT_SIZE)))
    x0p = catp.transpose(1, 0, 2).reshape(N, WID_PAD)

    r0, c0, v0 = _pad_edges(supp0_idx, supp0_val)
    r1, c1, v1 = _pad_edges(supp1_idx, supp1_val)
    eidx = jnp.stack([
        jnp.stack([r0, c0]), jnp.stack([r1, c1]),
    ])
    eidx = (eidx.reshape(2, 2, NTILES, NB, EB)
            .transpose(0, 2, 3, 1, 4)
            .reshape(2, NTILES, NWIN, W, 2, EB))
    evals = jnp.stack([v0, v1]).reshape(2, NTILES, NWIN, W, EB)

    zeros = jnp.zeros((N_PAD, F), jnp.float32)

    y1 = _spmm_shared(x0p, eidx, evals, zeros)
    t = _spmm_split(y1, eidx, evals, zeros)

    weight = mu_weight + jnp.exp(log_sigma_weight) * weight_noise
    bias = mu_biases + jnp.exp(log_sigma_biases) * bias_noise
    wr = weight.reshape(INPUT_SIZE, 5, OUT)
    wpad = lambda wm: jnp.pad(wm, ((0, ISP - INPUT_SIZE), (0, 0)))
    wcat = jnp.concatenate([
        wpad(wr[:, 0] - wr[:, 2] - wr[:, 4]),
        wpad(wr[:, 1]),
        wpad(2.0 * wr[:, 2]),
        wpad(wr[:, 3]),
        wpad(2.0 * wr[:, 4]),
    ], axis=0)

    def flat(a):
        return a.reshape(N_PAD * B, ISP)

    xs = [x0p.reshape(N * B, ISP),
          flat(y1[0]), flat(t[0]), flat(y1[1]), flat(t[1])]
    out = _matmul(xs, wcat, bias.reshape(1, OUT))
    return out.reshape(N, B, OUT).transpose(1, 0, 2).reshape(B, N * OUT)


# speedup vs baseline: 1.1599x; 1.1599x over previous
import functools

import jax
import jax.numpy as jnp
from jax import lax
from jax.experimental import pallas as pl
from jax.experimental.pallas import tpu as pltpu
from jax.experimental.pallas import tpu_sc as plsc

N = 10000
B = 32
IN_DIM = 2
HID = 32
OUT = 32
INPUT_SIZE = IN_DIM + HID
ISP = 36
WID_PAD = ISP * B
E = 160000
C = 9
F = WID_PAD // C
NTILES = 16
EB = 48
EPT = 10080
NB = EPT // EB
W = 42
NWIN = NB // W
NGRP = W // 3
E_PAD = EPT * NTILES
N_PAD = 10240
RPT = N_PAD // NTILES

_MESH = plsc.VectorSubcoreMesh(core_axis_name="c", subcore_axis_name="s")


def _scale(buf, evv, jl):
    def gi_body(gi, _):
        vv = evv[jl, pl.ds(16 * gi, 16)]
        for k in range(16):
            s = vv[k]
            e = 16 * gi + k
            for q in range(F // 16):
                buf[e, pl.ds(16 * q, 16)] = buf[e, pl.ds(16 * q, 16)] * s
        return 0
    lax.fori_loop(0, EB // 16, gi_body, 0)


def _make_spmm(shared_src: bool):
    @functools.partial(
        pl.kernel,
        mesh=_MESH,
        out_type=jax.ShapeDtypeStruct((2, N_PAD, WID_PAD), jnp.float32),
        scratch_types=[
            pltpu.VMEM((W, 2, EB), jnp.int32),
            pltpu.VMEM((W, EB), jnp.float32),
            pltpu.VMEM((EB, F), jnp.float32),
            pltpu.VMEM((EB, F), jnp.float32),
            pltpu.VMEM((EB, F), jnp.float32),
            pltpu.VMEM_SHARED((N_PAD, F), jnp.float32),
            pltpu.SemaphoreType.DMA,
            pltpu.SemaphoreType.DMA,
            pltpu.SemaphoreType.DMA,
            pltpu.SemaphoreType.DMA,
            pltpu.SemaphoreType.DMA,
            pltpu.SemaphoreType.DMA,
        ],
    )
    def spmm(src, eidx, evals, zeros, out,
             evw, evv, b0, b1, b2, acc, sg0, sg1, sg2, ss0, ss1, ss2):
        cid = lax.axis_index("c")
        sid = lax.axis_index("s")
        rowlo = sid * RPT
        bufs = (b0, b1, b2)
        sgs = (sg0, sg1, sg2)
        sss = (ss0, ss1, ss2)

        def chunk_body(c, _):
            coff = pl.multiple_of(c * F, F)
            if shared_src:
                src_c = src.at[:, pl.ds(coff, F)]
            else:
                src_c = src.at[cid, :, pl.ds(coff, F)]

            def start_gather(jl, slot):
                pltpu.async_copy(src_c.at[evw.at[jl, 1]], bufs[slot], sgs[slot])

            def wait_gather(jl, slot):
                pltpu.make_async_copy(
                    src_c.at[evw.at[jl, 1]], bufs[slot], sgs[slot]).wait()

            def start_scatter(jl, slot):
                pltpu.async_copy(bufs[slot], acc.at[evw.at[jl, 0]],
                                 sss[slot], add=True)

            def wait_scatter(jl, slot):
                pltpu.make_async_copy(
                    bufs[slot], acc.at[evw.at[jl, 0]], sss[slot]).wait()

            pltpu.sync_copy(zeros.at[pl.ds(rowlo, RPT)], acc.at[pl.ds(rowlo, RPT)])
            plsc.subcore_barrier()

            def win_body(w, _):
                pltpu.sync_copy(eidx.at[cid, sid, w], evw)
                pltpu.sync_copy(evals.at[cid, sid, w], evv)
                start_gather(0, 0)
                start_gather(1, 1)
                wait_gather(0, 0)
                _scale(b0, evv, 0)
                start_scatter(0, 0)
                start_gather(2, 2)
                wait_gather(1, 1)
                _scale(b1, evv, 1)
                start_scatter(1, 1)
                wait_scatter(0, 0)
                start_gather(3, 0)
                wait_gather(2, 2)
                _scale(b2, evv, 2)
                start_scatter(2, 2)

                def group_body(g, _):
                    for p in range(3):
                        jl = 3 * g + p
                        slot_next = (p + 1) % 3
                        wait_scatter(jl - 2, slot_next)
                        if p < 2:
                            start_gather(jl + 1, slot_next)
                        else:
                            @pl.when(g < NGRP - 1)
                            def _():
                                start_gather(jl + 1, slot_next)
                        wait_gather(jl, p)
                        _scale(bufs[p], evv, jl)
                        start_scatter(jl, p)
                    return 0

                lax.fori_loop(1, NGRP, group_body, 0)
                wait_scatter(W - 2, (W - 2) % 3)
                wait_scatter(W - 1, (W - 1) % 3)
                return 0

            lax.fori_loop(0, NWIN, win_body, 0)
            plsc.subcore_barrier()
            pltpu.sync_copy(acc.at[pl.ds(rowlo, RPT)],
                            out.at[cid, pl.ds(rowlo, RPT), pl.ds(coff, F)])
            plsc.subcore_barrier()
            return 0

        lax.fori_loop(0, C, chunk_body, 0)

    return spmm


_spmm_shared = _make_spmm(True)
_spmm_split = _make_spmm(False)

NBLK = 400


def _mm_body(x0r, x1r, x2r, x3r, x4r, wr, br, outr):
    acc = jnp.dot(x0r[...].astype(jnp.bfloat16), wr[0],
                  preferred_element_type=jnp.float32)
    acc += jnp.dot(x1r[...].astype(jnp.bfloat16), wr[1],
                   preferred_element_type=jnp.float32)
    acc += jnp.dot(x2r[...].astype(jnp.bfloat16), wr[2],
                   preferred_element_type=jnp.float32)
    acc += jnp.dot(x3r[...].astype(jnp.bfloat16), wr[3],
                   preferred_element_type=jnp.float32)
    acc += jnp.dot(x4r[...].astype(jnp.bfloat16), wr[4],
                   preferred_element_type=jnp.float32)
    outr[...] = acc + br[...]


def _matmul(xs, wbig, biasbig):
    x_spec = pl.BlockSpec((NBLK, WID_PAD), lambda i: (i, 0))
    return pl.pallas_call(
        _mm_body,
        grid=(N // NBLK,),
        in_specs=[x_spec] * 5 + [
            pl.BlockSpec((5, WID_PAD, B * OUT), lambda i: (0, 0, 0)),
            pl.BlockSpec((1, B * OUT), lambda i: (0, 0)),
        ],
        out_specs=pl.BlockSpec((NBLK, B * OUT), lambda i: (i, 0)),
        out_shape=jax.ShapeDtypeStruct((N, B * OUT), jnp.float32),
    )(*xs, wbig, biasbig)


def _pad_edges(idx, val):
    npad = E_PAD - E
    prow = N + (jnp.arange(npad, dtype=jnp.int32) % (N_PAD - N))
    pcol = jnp.arange(npad, dtype=jnp.int32) % N
    rows = jnp.concatenate([idx[0], prow])
    cols = jnp.concatenate([idx[1], pcol])
    vals = jnp.concatenate([val, jnp.zeros((npad,), jnp.float32)])
    return rows, cols, vals


def kernel(inputs, state, supp0_idx, supp0_val, supp1_idx, supp1_val,
           mu_weight, mu_biases, log_sigma_weight, log_sigma_biases,
           weight_noise, bias_noise):
    inp = inputs.reshape(B, N, IN_DIM)
    st = state.reshape(B, N, HID)
    cat = jnp.concatenate([inp, st], axis=2)
    catp = jnp.pad(cat, ((0, 0), (0, 0), (0, ISP - INPUT_SIZE)))
    x0p = catp.transpose(1, 0, 2).reshape(N, WID_PAD)

    r0, c0, v0 = _pad_edges(supp0_idx, supp0_val)
    r1, c1, v1 = _pad_edges(supp1_idx, supp1_val)
    eidx = jnp.stack([
        jnp.stack([r0, c0]), jnp.stack([r1, c1]),
    ])
    eidx = (eidx.reshape(2, 2, NTILES, NB, EB)
            .transpose(0, 2, 3, 1, 4)
            .reshape(2, NTILES, NWIN, W, 2, EB))
    evals = jnp.stack([v0, v1]).reshape(2, NTILES, NWIN, W, EB)

    zeros = jnp.zeros((N_PAD, F), jnp.float32)

    y1 = _spmm_shared(x0p, eidx, evals, zeros)
    t = _spmm_split(y1, eidx, evals, zeros)

    weight = mu_weight + jnp.exp(log_sigma_weight) * weight_noise
    bias = mu_biases + jnp.exp(log_sigma_biases) * bias_noise
    wr = weight.reshape(INPUT_SIZE, 5, OUT)
    wms = [wr[:, 0] - wr[:, 2] - wr[:, 4],
           wr[:, 1],
           2.0 * wr[:, 2],
           wr[:, 3],
           2.0 * wr[:, 4]]
    eye = jnp.eye(B, dtype=jnp.float32)
    wbig = jnp.stack([
        (eye[:, None, :, None] *
         jnp.pad(wm, ((0, ISP - INPUT_SIZE), (0, 0)))[None, :, None, :]
         ).reshape(WID_PAD, B * OUT)
        for wm in wms
    ]).astype(jnp.bfloat16)
    biasbig = jnp.tile(bias, B).reshape(1, B * OUT)

    xs = [x0p, y1[0], t[0], y1[1], t[1]]
    out = _matmul(xs, wbig, biasbig)
    return out.reshape(N, B, OUT).transpose(1, 0, 2).reshape(B, N * OUT)


# speedup vs baseline: 1.2049x; 1.0388x over previous
import functools

import jax
import jax.numpy as jnp
from jax import lax
from jax.experimental import pallas as pl
from jax.experimental.pallas import tpu as pltpu
from jax.experimental.pallas import tpu_sc as plsc

N = 10000
B = 32
IN_DIM = 2
HID = 32
OUT = 32
INPUT_SIZE = IN_DIM + HID
ISP = 36
WID_PAD = ISP * B
E = 160000
C = 9
F = WID_PAD // C
NTILES = 16
EB = 48
EPT = 10080
NB = EPT // EB
W = 42
NWIN = NB // W
NGRP = W // 3
E_PAD = EPT * NTILES
N_PAD = 10240
RPT = N_PAD // NTILES

_MESH = plsc.VectorSubcoreMesh(core_axis_name="c", subcore_axis_name="s")


def _scale(buf, evv, jl):
    def gi_body(gi, _):
        vv = evv[jl, pl.ds(16 * gi, 16)]
        for k in range(16):
            s = vv[k]
            e = 16 * gi + k
            for q in range(F // 16):
                buf[e, pl.ds(16 * q, 16)] = buf[e, pl.ds(16 * q, 16)] * s
        return 0
    lax.fori_loop(0, EB // 16, gi_body, 0)


def _make_spmm(shared_src: bool):
    @functools.partial(
        pl.kernel,
        mesh=_MESH,
        out_type=jax.ShapeDtypeStruct((2, N_PAD, WID_PAD), jnp.float32),
        scratch_types=[
            pltpu.VMEM((W, 2, EB), jnp.int32),
            pltpu.VMEM((W, EB), jnp.float32),
            pltpu.VMEM((EB, F), jnp.float32),
            pltpu.VMEM((EB, F), jnp.float32),
            pltpu.VMEM((EB, F), jnp.float32),
            pltpu.VMEM_SHARED((N_PAD, F), jnp.float32),
            pltpu.SemaphoreType.DMA,
            pltpu.SemaphoreType.DMA,
            pltpu.SemaphoreType.DMA,
            pltpu.SemaphoreType.DMA,
            pltpu.SemaphoreType.DMA,
            pltpu.SemaphoreType.DMA,
        ],
    )
    def spmm(src, eidx, evals, zeros, out,
             evw, evv, b0, b1, b2, acc, sg0, sg1, sg2, ss0, ss1, ss2):
        cid = lax.axis_index("c")
        sid = lax.axis_index("s")
        rowlo = sid * RPT
        bufs = (b0, b1, b2)
        sgs = (sg0, sg1, sg2)
        sss = (ss0, ss1, ss2)

        def chunk_body(c, _):
            coff = pl.multiple_of(c * F, F)
            if shared_src:
                src_c = src.at[:, pl.ds(coff, F)]
            else:
                src_c = src.at[cid, :, pl.ds(coff, F)]

            def start_gather(jl, slot):
                pltpu.async_copy(src_c.at[evw.at[jl, 1]], bufs[slot], sgs[slot])

            def wait_gather(jl, slot):
                pltpu.make_async_copy(
                    src_c.at[evw.at[jl, 1]], bufs[slot], sgs[slot]).wait()

            def start_scatter(jl, slot):
                pltpu.async_copy(bufs[slot], acc.at[evw.at[jl, 0]],
                                 sss[slot], add=True)

            def wait_scatter(jl, slot):
                pltpu.make_async_copy(
                    bufs[slot], acc.at[evw.at[jl, 0]], sss[slot]).wait()

            pltpu.sync_copy(zeros.at[pl.ds(rowlo, RPT)], acc.at[pl.ds(rowlo, RPT)])
            plsc.subcore_barrier()

            def win_body(w, _):
                pltpu.sync_copy(eidx.at[cid, sid, w], evw)
                pltpu.sync_copy(evals.at[cid, sid, w], evv)
                start_gather(0, 0)
                start_gather(1, 1)
                wait_gather(0, 0)
                _scale(b0, evv, 0)
                start_scatter(0, 0)
                start_gather(2, 2)
                wait_gather(1, 1)
                _scale(b1, evv, 1)
                start_scatter(1, 1)
                wait_scatter(0, 0)
                start_gather(3, 0)
                wait_gather(2, 2)
                _scale(b2, evv, 2)
                start_scatter(2, 2)

                def group_body(g, _):
                    for p in range(3):
                        jl = 3 * g + p
                        slot_next = (p + 1) % 3
                        wait_scatter(jl - 2, slot_next)
                        if p < 2:
                            start_gather(jl + 1, slot_next)
                        else:
                            @pl.when(g < NGRP - 1)
                            def _():
                                start_gather(jl + 1, slot_next)
                        wait_gather(jl, p)
                        _scale(bufs[p], evv, jl)
                        start_scatter(jl, p)
                    return 0

                lax.fori_loop(1, NGRP, group_body, 0)
                wait_scatter(W - 2, (W - 2) % 3)
                wait_scatter(W - 1, (W - 1) % 3)
                return 0

            lax.fori_loop(0, NWIN, win_body, 0)
            plsc.subcore_barrier()
            pltpu.sync_copy(acc.at[pl.ds(rowlo, RPT)],
                            out.at[cid, pl.ds(rowlo, RPT), pl.ds(coff, F)])
            plsc.subcore_barrier()
            return 0

        lax.fori_loop(0, C, chunk_body, 0)

    return spmm


_spmm_shared = _make_spmm(True)
_spmm_split = _make_spmm(False)

NBLK = 400


def _mm_body(x0r, x1r, x2r, x3r, x4r, wr, br, outr):
    acc = jnp.dot(x0r[...].astype(jnp.bfloat16), wr[0],
                  preferred_element_type=jnp.float32)
    acc += jnp.dot(x1r[...].astype(jnp.bfloat16), wr[1],
                   preferred_element_type=jnp.float32)
    acc += jnp.dot(x2r[...].astype(jnp.bfloat16), wr[2],
                   preferred_element_type=jnp.float32)
    acc += jnp.dot(x3r[...].astype(jnp.bfloat16), wr[3],
                   preferred_element_type=jnp.float32)
    acc += jnp.dot(x4r[...].astype(jnp.bfloat16), wr[4],
                   preferred_element_type=jnp.float32)
    accb = acc + br[...]
    outr[...] = (accb.reshape(NBLK, B, OUT).transpose(1, 0, 2)
                 .reshape(B, NBLK * OUT))


def _matmul(xs, wbig, biasbig):
    x_spec = pl.BlockSpec((NBLK, WID_PAD), lambda i: (i, 0))
    return pl.pallas_call(
        _mm_body,
        grid=(N // NBLK,),
        in_specs=[x_spec] * 5 + [
            pl.BlockSpec((5, WID_PAD, B * OUT), lambda i: (0, 0, 0)),
            pl.BlockSpec((1, B * OUT), lambda i: (0, 0)),
        ],
        out_specs=pl.BlockSpec((B, NBLK * OUT), lambda i: (0, i)),
        out_shape=jax.ShapeDtypeStruct((B, N * OUT), jnp.float32),
    )(*xs, wbig, biasbig)


def _pad_edges(idx, val):
    npad = E_PAD - E
    prow = N + (jnp.arange(npad, dtype=jnp.int32) % (N_PAD - N))
    pcol = jnp.arange(npad, dtype=jnp.int32) % N
    rows = jnp.concatenate([idx[0], prow])
    cols = jnp.concatenate([idx[1], pcol])
    vals = jnp.concatenate([val, jnp.zeros((npad,), jnp.float32)])
    return rows, cols, vals


def kernel(inputs, state, supp0_idx, supp0_val, supp1_idx, supp1_val,
           mu_weight, mu_biases, log_sigma_weight, log_sigma_biases,
           weight_noise, bias_noise):
    inp = inputs.reshape(B, N, IN_DIM)
    st = state.reshape(B, N, HID)
    cat = jnp.concatenate([inp, st], axis=2)
    catp = jnp.pad(cat, ((0, 0), (0, 0), (0, ISP - INPUT_SIZE)))
    x0p = catp.transpose(1, 0, 2).reshape(N, WID_PAD)

    r0, c0, v0 = _pad_edges(supp0_idx, supp0_val)
    r1, c1, v1 = _pad_edges(supp1_idx, supp1_val)
    eidx = jnp.stack([
        jnp.stack([r0, c0]), jnp.stack([r1, c1]),
    ])
    eidx = (eidx.reshape(2, 2, NTILES, NB, EB)
            .transpose(0, 2, 3, 1, 4)
            .reshape(2, NTILES, NWIN, W, 2, EB))
    evals = jnp.stack([v0, v1]).reshape(2, NTILES, NWIN, W, EB)

    zeros = jnp.zeros((N_PAD, F), jnp.float32)

    y1 = _spmm_shared(x0p, eidx, evals, zeros)
    t = _spmm_split(y1, eidx, evals, zeros)

    weight = mu_weight + jnp.exp(log_sigma_weight) * weight_noise
    bias = mu_biases + jnp.exp(log_sigma_biases) * bias_noise
    wr = weight.reshape(INPUT_SIZE, 5, OUT)
    wms = [wr[:, 0] - wr[:, 2] - wr[:, 4],
           wr[:, 1],
           2.0 * wr[:, 2],
           wr[:, 3],
           2.0 * wr[:, 4]]
    eye = jnp.eye(B, dtype=jnp.float32)
    wbig = jnp.stack([
        (eye[:, None, :, None] *
         jnp.pad(wm, ((0, ISP - INPUT_SIZE), (0, 0)))[None, :, None, :]
         ).reshape(WID_PAD, B * OUT)
        for wm in wms
    ]).astype(jnp.bfloat16)
    biasbig = jnp.tile(bias, B).reshape(1, B * OUT)

    xs = [x0p, y1[0], t[0], y1[1], t[1]]
    return _matmul(xs, wbig, biasbig)
